# baseline (speedup 1.0000x reference)
import functools

import jax
import jax.numpy as jnp
from jax import lax
from jax.experimental import pallas as pl
from jax.experimental.pallas import tpu as pltpu
from jax.experimental.pallas import tpu_sc as plsc

BATCH = 16384
D = 32
NUM_CORES = 2
NUM_SUBCORES = 16
NW = NUM_CORES * NUM_SUBCORES
B_PER_W = BATCH // NW
CHUNK = 128
NCHUNK = B_PER_W // CHUNK
LANES = 16


def _body(labels_hbm, mu_hbm, sigma_hbm, z_hbm, out_hbm,
          idx_v, mu_v, sigma_v, z_v, sem):
    wid = lax.axis_index("s") * NUM_CORES + lax.axis_index("c")
    base = wid * B_PER_W

    pltpu.sync_copy(labels_hbm.at[pl.ds(base, B_PER_W)], idx_v)

    copies = []
    for c in range(NCHUNK):
        sl = pl.ds(c * CHUNK, CHUNK)
        copies.append(pltpu.async_copy(mu_hbm.at[idx_v.at[sl]], mu_v.at[sl], sem))
        copies.append(pltpu.async_copy(sigma_hbm.at[idx_v.at[sl]], sigma_v.at[sl], sem))
    pltpu.sync_copy(z_hbm.at[pl.ds(base, B_PER_W)], z_v)
    for cp in copies:
        cp.wait()

    def row(i, _):
        for h in range(D // LANES):
            sl = pl.ds(h * LANES, LANES)
            s = sigma_v[i, sl]
            z_v[i, sl] = z_v[i, sl] * jnp.exp(0.5 * s) + mu_v[i, sl]
        return 0

    lax.fori_loop(0, B_PER_W, row, 0, unroll=4)

    pltpu.sync_copy(z_v, out_hbm.at[pl.ds(base, B_PER_W)])


@functools.partial(jax.jit, donate_argnums=())
def kernel(labels, mu_table, sigma_table, z):
    mesh = plsc.VectorSubcoreMesh(core_axis_name="c", subcore_axis_name="s")
    k = functools.partial(
        pl.kernel,
        mesh=mesh,
        out_type=jax.ShapeDtypeStruct((BATCH, D), jnp.float32),
        scratch_types=[
            pltpu.VMEM((B_PER_W,), jnp.int32),
            pltpu.VMEM((B_PER_W, D), jnp.float32),
            pltpu.VMEM((B_PER_W, D), jnp.float32),
            pltpu.VMEM((B_PER_W, D), jnp.float32),
            pltpu.SemaphoreType.DMA,
        ],
        compiler_params=pltpu.CompilerParams(use_tc_tiling_on_sc=False),
    )(_body)
    return k(labels.astype(jnp.int32), mu_table, sigma_table, z)
